# baseline (speedup 1.0000x reference)
import jax
import jax.numpy as jnp
from jax.experimental import pallas as pl
from jax.experimental.pallas import tpu as pltpu

_BM = 200


def _epilogue(out, b):
    h = jnp.maximum(out + b, 0.0)
    m = jnp.max(h, axis=1, keepdims=True)
    lse = m + jnp.log(jnp.sum(jnp.exp(h - m), axis=1, keepdims=True))
    return h - lse


def _gcn_block_kernel(x_ref, adjt_ref, adjb_ref, w_ref, b_ref, out_ref,
                      support_ref):
    i = pl.program_id(0)
    nsteps = pl.num_programs(0)

    @pl.when(i == 0)
    def _():
        support_ref[...] = jnp.dot(
            x_ref[...], w_ref[...], preferred_element_type=jnp.float32
        )

    b = b_ref[...]
    out_t = jnp.dot(
        adjt_ref[...], support_ref[...], preferred_element_type=jnp.float32
    )
    out_ref[pl.ds(i * _BM, _BM), :] = _epilogue(out_t, b)
    out_b = jnp.dot(
        adjb_ref[...], support_ref[...], preferred_element_type=jnp.float32
    )
    out_ref[pl.ds((nsteps + i) * _BM, _BM), :] = _epilogue(out_b, b)


def kernel(x, adj, W, b):
    n, feat = x.shape
    nclass = W.shape[1]
    b2 = b.reshape(1, nclass)
    nsteps = n // (2 * _BM)
    return pl.pallas_call(
        _gcn_block_kernel,
        grid=(nsteps,),
        in_specs=[
            pl.BlockSpec((n, feat), lambda i: (0, 0)),
            pl.BlockSpec((_BM, n), lambda i: (i, 0)),
            pl.BlockSpec((_BM, n), lambda i: (i + n // (2 * _BM), 0)),
            pl.BlockSpec((feat, nclass), lambda i: (0, 0)),
            pl.BlockSpec((1, nclass), lambda i: (0, 0)),
        ],
        out_specs=pl.BlockSpec((n, nclass), lambda i: (0, 0)),
        out_shape=jax.ShapeDtypeStruct((n, nclass), jnp.float32),
        scratch_shapes=[pltpu.VMEM((n, nclass), jnp.float32)],
        compiler_params=pltpu.CompilerParams(
            vmem_limit_bytes=64 * 1024 * 1024,
        ),
    )(x, adj, adj, W, b2)
